# baseline (speedup 1.0000x reference)
import functools

import jax
import jax.numpy as jnp
from jax import lax
from jax.experimental import pallas as pl
from jax.experimental.pallas import tpu as pltpu
from jax.experimental.pallas import tpu_sc as plsc

_NC = 2
_NS = 16
_NW = _NC * _NS
_H = 16

_B_TC = 8
_S_BLK = 256


def _sc_body(ids_hbm, wte_hbm, wpe_hbm, out_hbm,
             idx_all, wpe_v, gbuf0, gbuf1, obuf0, obuf1,
             gs0, gs1, ws0, ws1, isem):
    B, _ = ids_hbm.shape
    P, D = wpe_v.shape
    wid = lax.axis_index("s") * _NC + lax.axis_index("c")
    p0 = wid * P
    idx_copies = [
        pltpu.make_async_copy(ids_hbm.at[b, pl.ds(p0, P)], idx_all.at[b], isem)
        for b in range(B)
    ]
    for c in idx_copies:
        c.start()
    pltpu.sync_copy(wpe_hbm.at[pl.ds(p0, P)], wpe_v)
    for c in idx_copies:
        c.wait()

    gbufs = (gbuf0, gbuf1)
    obufs = (obuf0, obuf1)
    gsems = (gs0, gs1)
    wsems = (ws0, ws1)

    def gather(k, r):
        idx = idx_all.at[k, pl.ds(r * _H, _H)]
        return pltpu.make_async_copy(wte_hbm.at[idx], gbufs[r], gsems[r])

    def write(k, r):
        dst = out_hbm.at[k, pl.ds(p0 + r * _H, _H)]
        return pltpu.make_async_copy(obufs[r], dst, wsems[r])

    def add_rows(r):
        g, o = gbufs[r], obufs[r]

        def row_body(i, c):
            for j in range(D // 16):
                sl = pl.ds(j * 16, 16)
                o[i, sl] = g[i, sl] + wpe_v[r * _H + i, sl]
            return c

        lax.fori_loop(0, _H, row_body, 0)

    gather(0, 0).start()
    gather(0, 1).start()

    def batch_body(k, carry):
        for r in range(2):
            gather(k, r).wait()

            @pl.when(k > 0)
            def _():
                write(k - 1, r).wait()

            add_rows(r)

            @pl.when(k < B - 1)
            def _():
                gather(k + 1, r).start()

            write(k, r).start()

        return carry

    lax.fori_loop(0, B, batch_body, 0)
    write(B - 1, 0).wait()
    write(B - 1, 1).wait()


def _sc_part(ids, wte, wpe):
    B, S = ids.shape
    V, D = wte.shape
    P = S // _NW
    mesh = plsc.VectorSubcoreMesh(
        core_axis_name="c", subcore_axis_name="s",
        num_cores=_NC, num_subcores=_NS,
    )
    f = pl.kernel(
        _sc_body,
        out_type=jax.ShapeDtypeStruct((B, S, D), jnp.float32),
        mesh=mesh,
        scratch_types=[
            pltpu.VMEM((B, P), jnp.int32),
            pltpu.VMEM((P, D), jnp.float32),
            pltpu.VMEM((_H, D), jnp.float32),
            pltpu.VMEM((_H, D), jnp.float32),
            pltpu.VMEM((_H, D), jnp.float32),
            pltpu.VMEM((_H, D), jnp.float32),
            pltpu.SemaphoreType.DMA,
            pltpu.SemaphoreType.DMA,
            pltpu.SemaphoreType.DMA,
            pltpu.SemaphoreType.DMA,
            pltpu.SemaphoreType.DMA,
        ],
    )
    return f(ids, wte, wpe)


def _tc_body(ids_ref, wte_ref, wpe_ref, out_ref, rows, sem):
    def fetch(i, c):
        idx = ids_ref[0, 0, i]
        pltpu.make_async_copy(wte_ref.at[idx], rows.at[i], sem).start()
        return c

    lax.fori_loop(0, _S_BLK, fetch, 0)

    def drain(i, c):
        pltpu.make_async_copy(wte_ref.at[0], rows.at[i], sem).wait()
        return c

    lax.fori_loop(0, _S_BLK, drain, 0)
    out_ref[0] = rows[...] + wpe_ref[...]


def _tc_part(ids, wte, wpe):
    B, S = ids.shape
    V, D = wte.shape
    nblk = S // _S_BLK
    ids3 = ids.reshape(B * nblk, 1, _S_BLK)
    grid = (nblk, B)
    return pl.pallas_call(
        _tc_body,
        grid=grid,
        in_specs=[
            pl.BlockSpec((1, 1, _S_BLK), lambda s, b, n=nblk: (b * n + s, 0, 0),
                         memory_space=pltpu.SMEM),
            pl.BlockSpec(memory_space=pl.ANY),
            pl.BlockSpec((_S_BLK, D), lambda s, b: (s, 0)),
        ],
        out_specs=pl.BlockSpec((1, _S_BLK, D), lambda s, b: (b, s, 0)),
        out_shape=jax.ShapeDtypeStruct((B, S, D), jnp.float32),
        scratch_shapes=[
            pltpu.VMEM((_S_BLK, D), jnp.float32),
            pltpu.SemaphoreType.DMA,
        ],
    )(ids3, wte, wpe)


def kernel(input_ids, wte, wpe):
    ids = input_ids.astype(jnp.int32)
    B, S = ids.shape
    b_sc = B - _B_TC
    out_sc = _sc_part(ids[:b_sc], wte, wpe)
    out_tc = _tc_part(ids[b_sc:], wte, wpe)
    return jnp.concatenate([out_sc, out_tc], axis=0)


# speedup vs baseline: 2.0552x; 2.0552x over previous
import jax
import jax.numpy as jnp
from jax import lax
from jax.experimental import pallas as pl
from jax.experimental.pallas import tpu as pltpu
from jax.experimental.pallas import tpu_sc as plsc

_NC = 2
_NS = 16
_NW = _NC * _NS
_H = 16


def _emb_body(ids_hbm, wte_hbm, wpe_hbm, out_hbm,
              idx_all, wpe_v, gbuf0, gbuf1, obuf0, obuf1,
              gs0, gs1, ws0, ws1, isem):
    B, _ = ids_hbm.shape
    P, D = wpe_v.shape
    wid = lax.axis_index("s") * _NC + lax.axis_index("c")
    p0 = wid * P
    idx_copies = [
        pltpu.make_async_copy(ids_hbm.at[b, pl.ds(p0, P)], idx_all.at[b], isem)
        for b in range(B)
    ]
    for c in idx_copies:
        c.start()
    pltpu.sync_copy(wpe_hbm.at[pl.ds(p0, P)], wpe_v)
    for c in idx_copies:
        c.wait()

    gbufs = (gbuf0, gbuf1)
    obufs = (obuf0, obuf1)
    gsems = (gs0, gs1)
    wsems = (ws0, ws1)

    def gather(k, r):
        idx = idx_all.at[k, pl.ds(r * _H, _H)]
        return pltpu.make_async_copy(wte_hbm.at[idx], gbufs[r], gsems[r])

    def write(k, r):
        dst = out_hbm.at[k, pl.ds(p0 + r * _H, _H)]
        return pltpu.make_async_copy(obufs[r], dst, wsems[r])

    def add_rows(r):
        g, o = gbufs[r], obufs[r]

        def row_body(i, c):
            for j in range(D // 16):
                sl = pl.ds(j * 16, 16)
                o[i, sl] = g[i, sl] + wpe_v[r * _H + i, sl]
            return c

        lax.fori_loop(0, _H, row_body, 0)

    gather(0, 0).start()
    gather(0, 1).start()

    def batch_body(k, carry):
        for r in range(2):
            gather(k, r).wait()

            @pl.when(k > 0)
            def _():
                write(k - 1, r).wait()

            add_rows(r)
            write(k, r).start()

            @pl.when(k < B - 1)
            def _():
                gather(k + 1, r).start()

        return carry

    lax.fori_loop(0, B, batch_body, 0)
    write(B - 1, 0).wait()
    write(B - 1, 1).wait()


def kernel(input_ids, wte, wpe):
    B, S = input_ids.shape
    V, D = wte.shape
    P = S // _NW
    mesh = plsc.VectorSubcoreMesh(
        core_axis_name="c", subcore_axis_name="s",
        num_cores=_NC, num_subcores=_NS,
    )
    f = pl.kernel(
        _emb_body,
        out_type=jax.ShapeDtypeStruct((B, S, D), jnp.float32),
        mesh=mesh,
        scratch_types=[
            pltpu.VMEM((B, P), jnp.int32),
            pltpu.VMEM((P, D), jnp.float32),
            pltpu.VMEM((_H, D), jnp.float32),
            pltpu.VMEM((_H, D), jnp.float32),
            pltpu.VMEM((_H, D), jnp.float32),
            pltpu.VMEM((_H, D), jnp.float32),
            pltpu.SemaphoreType.DMA,
            pltpu.SemaphoreType.DMA,
            pltpu.SemaphoreType.DMA,
            pltpu.SemaphoreType.DMA,
            pltpu.SemaphoreType.DMA,
        ],
    )
    return f(input_ids.astype(jnp.int32), wte, wpe)
